# baseline (speedup 1.0000x reference)
import functools

import jax
import jax.numpy as jnp
from jax import lax
from jax.experimental import pallas as pl
from jax.experimental.pallas import tpu as pltpu
from jax.experimental.pallas import tpu_sc as plsc

N_NODES = 10000
D = 128
E = 320000
NC, NS = 2, 16
NW = NC * NS
N_PAD = 10240
RPT = N_PAD // NS
CH = 128
GSZ = 8
NGRP = 10
NCHUNK = NGRP * GSZ
EPT = NCHUNK * CH
E_PAD = NW * EPT
XT_PAD = N_NODES + 16
WCH = 128

_MESH = plsc.VectorSubcoreMesh(
    core_axis_name="c", subcore_axis_name="s", num_cores=NC, num_subcores=NS
)


@functools.partial(
    pl.kernel,
    out_type=jax.ShapeDtypeStruct((NC, N_PAD), jnp.float32),
    mesh=_MESH,
    scratch_types=[
        pltpu.VMEM((NCHUNK, CH), jnp.int32),
        pltpu.VMEM((CH,), jnp.float32),
        pltpu.VMEM((RPT,), jnp.float32),
        pltpu.VMEM_SHARED((N_PAD,), jnp.float32),
        pltpu.SemaphoreType.DMA,
    ],
)
def _deg_kernel(dst2_hbm, ones_hbm, zrow_hbm, degp_hbm, dsts_v, ones_v,
                buf_v, deg_sh, ssem):
    c = lax.axis_index("c")
    s = lax.axis_index("s")
    rbase = pl.multiple_of((c * NS + s) * NCHUNK, 8)
    pltpu.sync_copy(dst2_hbm.at[pl.ds(rbase, NCHUNK)], dsts_v)
    pltpu.sync_copy(ones_hbm, ones_v)
    pltpu.sync_copy(zrow_hbm, buf_v)
    row0 = pl.multiple_of(s * RPT, 8)
    pltpu.sync_copy(buf_v, deg_sh.at[pl.ds(row0, RPT)])
    plsc.subcore_barrier()

    def outer(k, carry):
        for jj in range(16):
            pltpu.async_copy(ones_v, deg_sh.at[dsts_v.at[k * 16 + jj]], ssem,
                             add=True)
        for jj in range(16):
            pltpu.make_async_copy(
                ones_v, deg_sh.at[dsts_v.at[k * 16 + jj]], ssem).wait()
        return carry

    lax.fori_loop(0, NCHUNK // 16, outer, 0)
    plsc.subcore_barrier()
    pltpu.sync_copy(deg_sh.at[pl.ds(row0, RPT)], buf_v)
    pltpu.sync_copy(buf_v, degp_hbm.at[c, pl.ds(row0, RPT)])


@functools.partial(
    pl.kernel,
    out_type=jax.ShapeDtypeStruct((NC, N_PAD, D), jnp.float32),
    mesh=_MESH,
    scratch_types=[
        pltpu.VMEM((GSZ, CH), jnp.int32),
        pltpu.VMEM((GSZ, CH), jnp.int32),
        pltpu.VMEM((GSZ, CH), jnp.int32),
        pltpu.VMEM((GSZ, CH), jnp.int32),
        pltpu.VMEM((CH, D), jnp.float32),
        pltpu.VMEM((CH, D), jnp.float32),
        pltpu.VMEM_SHARED((N_PAD, D), jnp.float32),
        pltpu.SemaphoreType.DMA,
        pltpu.SemaphoreType.DMA,
        pltpu.SemaphoreType.DMA,
        pltpu.SemaphoreType.DMA,
        pltpu.SemaphoreType.DMA,
        pltpu.SemaphoreType.DMA,
    ],
)
def _sweep_kernel(src2_hbm, dst2_hbm, xt_hbm, zrows_hbm, q_hbm,
                  ibs0, ibd0, ibs1, ibd1, rowsA, rowsB, out_sh,
                  is0, is1, gsA, gsB, ssA, ssB):
    c = lax.axis_index("c")
    s = lax.axis_index("s")
    rbase = pl.multiple_of((c * NS + s) * NCHUNK, 8)
    pltpu.sync_copy(zrows_hbm, rowsA)
    for j in range(RPT // WCH):
        r0 = pl.multiple_of(s * RPT + j * WCH, 8)
        pltpu.sync_copy(rowsA, out_sh.at[pl.ds(r0, WCH)])

    def istart(g, ibs, ibd, sem):
        r = pl.multiple_of(rbase + g * GSZ, 8)
        pltpu.async_copy(src2_hbm.at[pl.ds(r, GSZ)], ibs, sem)
        pltpu.async_copy(dst2_hbm.at[pl.ds(r, GSZ)], ibd, sem)

    def iwait(ibs, ibd, sem):
        pltpu.make_async_copy(src2_hbm.at[pl.ds(rbase, GSZ)], ibs, sem).wait()
        pltpu.make_async_copy(dst2_hbm.at[pl.ds(rbase, GSZ)], ibd, sem).wait()

    def gstart(ib, jj, rows, sem):
        pltpu.async_copy(xt_hbm.at[ib.at[jj]], rows, sem)

    def gwait(ib, jj, rows, sem):
        pltpu.make_async_copy(xt_hbm.at[ib.at[jj]], rows, sem).wait()

    def sstart(ib, jj, rows, sem):
        pltpu.async_copy(rows, out_sh.at[ib.at[jj]], sem, add=True)

    def swait(ib, jj, rows, sem):
        pltpu.make_async_copy(rows, out_sh.at[ib.at[jj]], sem).wait()

    istart(0, ibs0, ibd0, is0)
    istart(1, ibs1, ibd1, is1)
    plsc.subcore_barrier()

    def group(g, ibs, ibd, isem, nxt, first):
        iwait(ibs, ibd, isem)
        if first:
            gstart(ibs, 0, rowsA, gsA)
            gstart(ibs, 1, rowsB, gsB)
        else:
            swait(ibd, 0, rowsA, ssA)
            gstart(ibs, 0, rowsA, gsA)
            swait(ibd, 1, rowsB, ssB)
            gstart(ibs, 1, rowsB, gsB)
        if nxt is not None:
            istart(*nxt)
        gwait(ibs, 0, rowsA, gsA)
        sstart(ibd, 0, rowsA, ssA)
        gwait(ibs, 1, rowsB, gsB)
        sstart(ibd, 1, rowsB, ssB)
        for m in range(1, GSZ // 2):
            jj = 2 * m
            swait(ibd, jj, rowsA, ssA)
            gstart(ibs, jj, rowsA, gsA)
            swait(ibd, jj + 1, rowsB, ssB)
            gstart(ibs, jj + 1, rowsB, gsB)
            gwait(ibs, jj, rowsA, gsA)
            sstart(ibd, jj, rowsA, ssA)
            gwait(ibs, jj + 1, rowsB, gsB)
            sstart(ibd, jj + 1, rowsB, ssB)

    group(0, ibs0, ibd0, is0, None, first=True)

    def body(gg, carry):
        g = 2 * gg + 1
        group(g, ibs1, ibd1, is1, (g + 1, ibs0, ibd0, is0), first=False)
        group(g + 1, ibs0, ibd0, is0, (g + 2, ibs1, ibd1, is1), first=False)
        return carry

    lax.fori_loop(0, (NGRP - 2) // 2, body, 0)
    group(NGRP - 1, ibs1, ibd1, is1, None, first=False)
    swait(ibd1, GSZ - 2, rowsA, ssA)
    swait(ibd1, GSZ - 1, rowsB, ssB)
    plsc.subcore_barrier()

    for j in range(RPT // WCH):
        r0 = pl.multiple_of(s * RPT + j * WCH, 8)
        rows = rowsA if j % 2 == 0 else rowsB
        pltpu.sync_copy(out_sh.at[pl.ds(r0, WCH)], rows)
        pltpu.sync_copy(rows, q_hbm.at[c, pl.ds(r0, WCH)])


def _mm_body(feats_ref, w_ref, d0_ref, d1_ref, xt_ref, dinv_ref):
    deg = d0_ref[...] + d1_ref[...]
    good = deg > 0.0
    dinv = jnp.where(good, lax.rsqrt(jnp.where(good, deg, 1.0)), 0.0)
    x = jnp.dot(feats_ref[...], w_ref[...], preferred_element_type=jnp.float32)
    xt_ref[:N_NODES, :] = x * dinv
    xt_ref[N_NODES:, :] = jnp.zeros((XT_PAD - N_NODES, D), jnp.float32)
    dinv_ref[...] = dinv


def _fin_body(q_ref, dinv_ref, b_ref, o_ref):
    acc = q_ref[0, :N_NODES, :] + q_ref[1, :N_NODES, :]
    o_ref[...] = acc * dinv_ref[...] + b_ref[...]


def kernel(feats, edges, W, b):
    ei = edges.astype(jnp.int32)
    ei = jnp.pad(ei, ((0, 0), (0, E_PAD - E)), constant_values=N_NODES)
    src2 = ei[0].reshape(E_PAD // CH, CH)
    dst2 = ei[1].reshape(E_PAD // CH, CH)
    ones_c = jnp.ones((CH,), jnp.float32)
    zrow = jnp.zeros((RPT,), jnp.float32)
    zrows = jnp.zeros((CH, D), jnp.float32)

    degp = _deg_kernel(dst2, ones_c, zrow)
    deg0 = degp[0, :N_NODES, None]
    deg1 = degp[1, :N_NODES, None]

    xt, dinv = pl.pallas_call(
        _mm_body,
        out_shape=(
            jax.ShapeDtypeStruct((XT_PAD, D), jnp.float32),
            jax.ShapeDtypeStruct((N_NODES, 1), jnp.float32),
        ),
    )(feats, W, deg0, deg1)

    q = _sweep_kernel(src2, dst2, xt, zrows)

    out = pl.pallas_call(
        _fin_body,
        out_shape=jax.ShapeDtypeStruct((N_NODES, D), jnp.float32),
    )(q, dinv, b.reshape(1, D))
    return out


# speedup vs baseline: 1.0374x; 1.0374x over previous
import functools

import jax
import jax.numpy as jnp
from jax import lax
from jax.experimental import pallas as pl
from jax.experimental.pallas import tpu as pltpu
from jax.experimental.pallas import tpu_sc as plsc

N_NODES = 10000
D = 128
E = 320000
NC, NS = 2, 16
NW = NC * NS
N_PAD = 10240
RPT = N_PAD // NS
CH = 80
GSZ = 8
NGRP = 16
NCHUNK = NGRP * GSZ
EPT = NCHUNK * CH
E_PAD = NW * EPT
XT_PAD = N_NODES + 16
WCH = 80

_MESH = plsc.VectorSubcoreMesh(
    core_axis_name="c", subcore_axis_name="s", num_cores=NC, num_subcores=NS
)


@functools.partial(
    pl.kernel,
    out_type=jax.ShapeDtypeStruct((NC, N_PAD), jnp.float32),
    mesh=_MESH,
    scratch_types=[
        pltpu.VMEM((NCHUNK, CH), jnp.int32),
        pltpu.VMEM((CH,), jnp.float32),
        pltpu.VMEM((RPT,), jnp.float32),
        pltpu.VMEM_SHARED((N_PAD,), jnp.float32),
        pltpu.SemaphoreType.DMA,
    ],
)
def _deg_kernel(dst2_hbm, ones_hbm, zrow_hbm, degp_hbm, dsts_v, ones_v,
                buf_v, deg_sh, ssem):
    c = lax.axis_index("c")
    s = lax.axis_index("s")
    rbase = pl.multiple_of((c * NS + s) * NCHUNK, 8)
    pltpu.sync_copy(dst2_hbm.at[pl.ds(rbase, NCHUNK)], dsts_v)
    pltpu.sync_copy(ones_hbm, ones_v)
    pltpu.sync_copy(zrow_hbm, buf_v)
    row0 = pl.multiple_of(s * RPT, 8)
    pltpu.sync_copy(buf_v, deg_sh.at[pl.ds(row0, RPT)])
    plsc.subcore_barrier()

    def outer(k, carry):
        for jj in range(16):
            pltpu.async_copy(ones_v, deg_sh.at[dsts_v.at[k * 16 + jj]], ssem,
                             add=True)
        for jj in range(16):
            pltpu.make_async_copy(
                ones_v, deg_sh.at[dsts_v.at[k * 16 + jj]], ssem).wait()
        return carry

    lax.fori_loop(0, NCHUNK // 16, outer, 0)
    plsc.subcore_barrier()
    pltpu.sync_copy(deg_sh.at[pl.ds(row0, RPT)], buf_v)
    pltpu.sync_copy(buf_v, degp_hbm.at[c, pl.ds(row0, RPT)])


NBUF = 4
NSLAB = 2
GLAG = 2


@functools.partial(
    pl.kernel,
    out_type=jax.ShapeDtypeStruct((NC, N_PAD, D), jnp.float32),
    mesh=_MESH,
    scratch_types=(
        [pltpu.VMEM((GSZ, CH), jnp.int32)] * (2 * NSLAB)
        + [pltpu.VMEM((CH, D), jnp.float32)] * NBUF
        + [pltpu.VMEM_SHARED((N_PAD, D), jnp.float32)]
        + [pltpu.SemaphoreType.DMA] * (NSLAB + 2 * NBUF)
    ),
)
def _sweep_kernel(src2_hbm, dst2_hbm, xt_hbm, zrows_hbm, q_hbm, *refs):
    ibs = list(refs[0:NSLAB])
    ibd = list(refs[NSLAB:2 * NSLAB])
    rows = list(refs[2 * NSLAB:2 * NSLAB + NBUF])
    out_sh = refs[2 * NSLAB + NBUF]
    isem = list(refs[2 * NSLAB + NBUF + 1:2 * NSLAB + NBUF + 1 + NSLAB])
    gsem = list(refs[2 * NSLAB + NBUF + 1 + NSLAB:
                     2 * NSLAB + NBUF + 1 + NSLAB + NBUF])
    ssem = list(refs[2 * NSLAB + NBUF + 1 + NSLAB + NBUF:])
    c = lax.axis_index("c")
    s = lax.axis_index("s")
    rbase = pl.multiple_of((c * NS + s) * NCHUNK, 8)

    def istart(g):
        p = g % NSLAB
        r = pl.multiple_of(rbase + g * GSZ, 8)
        pltpu.async_copy(src2_hbm.at[pl.ds(r, GSZ)], ibs[p], isem[p])
        pltpu.async_copy(dst2_hbm.at[pl.ds(r, GSZ)], ibd[p], isem[p])

    def iwait(g):
        p = g % NSLAB
        pltpu.make_async_copy(
            src2_hbm.at[pl.ds(rbase, GSZ)], ibs[p], isem[p]).wait()
        pltpu.make_async_copy(
            dst2_hbm.at[pl.ds(rbase, GSZ)], ibd[p], isem[p]).wait()

    def gstart(t):
        p, jj, b = (t // GSZ) % NSLAB, t % GSZ, t % NBUF
        pltpu.async_copy(xt_hbm.at[ibs[p].at[jj]], rows[b], gsem[b])

    def gwait(t):
        p, jj, b = (t // GSZ) % NSLAB, t % GSZ, t % NBUF
        pltpu.make_async_copy(
            xt_hbm.at[ibs[p].at[jj]], rows[b], gsem[b]).wait()

    def sstart(t):
        p, jj, b = (t // GSZ) % NSLAB, t % GSZ, t % NBUF
        pltpu.async_copy(rows[b], out_sh.at[ibd[p].at[jj]], ssem[b], add=True)

    def swait(t):
        p, jj, b = (t // GSZ) % NSLAB, t % GSZ, t % NBUF
        pltpu.make_async_copy(rows[b], out_sh.at[ibd[p].at[jj]], ssem[b]).wait()

    pltpu.sync_copy(zrows_hbm, rows[0])
    nz = RPT // WCH
    for j in range(nz):
        r0 = pl.multiple_of(s * RPT + j * WCH, 8)
        pltpu.async_copy(rows[0], out_sh.at[pl.ds(r0, WCH)], ssem[j % NBUF])
    for j in range(nz):
        r0 = pl.multiple_of(s * RPT + j * WCH, 8)
        pltpu.make_async_copy(
            rows[0], out_sh.at[pl.ds(r0, WCH)], ssem[j % NBUF]).wait()
    for g in range(NSLAB):
        istart(g)
    plsc.subcore_barrier()

    for t in range(NCHUNK + NBUF):
        if t >= NBUF and t - NBUF < NCHUNK:
            swait(t - NBUF)
        if t < NCHUNK:
            g, jj = t // GSZ, t % GSZ
            if jj == 0:
                iwait(g)
            gstart(t)
            if jj == NBUF and 1 <= g <= NGRP - NSLAB:
                istart(g + NSLAB - 1)
        if t >= GLAG and t - GLAG < NCHUNK:
            gwait(t - GLAG)
            sstart(t - GLAG)
    plsc.subcore_barrier()

    def _wr_r0(j):
        return pl.multiple_of(s * RPT + j * WCH, 8)

    for j in range(nz + NBUF):
        b = j % NBUF
        if NBUF <= j:
            jp = j - NBUF
            if jp < nz:
                pltpu.make_async_copy(
                    rows[b], q_hbm.at[c, pl.ds(_wr_r0(jp), WCH)],
                    ssem[b]).wait()
        if j < nz:
            pltpu.async_copy(out_sh.at[pl.ds(_wr_r0(j), WCH)], rows[b],
                             gsem[b])
            pltpu.make_async_copy(
                out_sh.at[pl.ds(_wr_r0(j), WCH)], rows[b], gsem[b]).wait()
            pltpu.async_copy(rows[b], q_hbm.at[c, pl.ds(_wr_r0(j), WCH)],
                             ssem[b])


def _mm_body(feats_ref, w_ref, d0_ref, d1_ref, xt_ref, dinv_ref):
    deg = d0_ref[...] + d1_ref[...]
    good = deg > 0.0
    dinv = jnp.where(good, lax.rsqrt(jnp.where(good, deg, 1.0)), 0.0)
    x = jnp.dot(feats_ref[...], w_ref[...], preferred_element_type=jnp.float32)
    xt_ref[:N_NODES, :] = x * dinv
    xt_ref[N_NODES:, :] = jnp.zeros((XT_PAD - N_NODES, D), jnp.float32)
    dinv_ref[...] = dinv


def _fin_body(q_ref, dinv_ref, b_ref, o_ref):
    acc = q_ref[0, :N_NODES, :] + q_ref[1, :N_NODES, :]
    o_ref[...] = acc * dinv_ref[...] + b_ref[...]


def kernel(feats, edges, W, b):
    ei = edges.astype(jnp.int32)
    ei = jnp.pad(ei, ((0, 0), (0, E_PAD - E)), constant_values=N_NODES)
    src2 = ei[0].reshape(E_PAD // CH, CH)
    dst2 = ei[1].reshape(E_PAD // CH, CH)
    ones_c = jnp.ones((CH,), jnp.float32)
    zrow = jnp.zeros((RPT,), jnp.float32)
    zrows = jnp.zeros((CH, D), jnp.float32)

    degp = _deg_kernel(dst2, ones_c, zrow)
    deg0 = degp[0, :N_NODES, None]
    deg1 = degp[1, :N_NODES, None]

    xt, dinv = pl.pallas_call(
        _mm_body,
        out_shape=(
            jax.ShapeDtypeStruct((XT_PAD, D), jnp.float32),
            jax.ShapeDtypeStruct((N_NODES, 1), jnp.float32),
        ),
    )(feats, W, deg0, deg1)

    q = _sweep_kernel(src2, dst2, xt, zrows)

    out = pl.pallas_call(
        _fin_body,
        out_shape=jax.ShapeDtypeStruct((N_NODES, D), jnp.float32),
    )(q, dinv, b.reshape(1, D))
    return out
